# baseline (speedup 1.0000x reference)
import functools

import jax
import jax.numpy as jnp
from jax import lax
from jax.experimental import pallas as pl
from jax.experimental.pallas import tpu as pltpu
from jax.experimental.pallas import tpu_sc as plsc

_NC = 2
_NS = 16
_NW = _NC * _NS
_L = 16


def _sc_count_body(p_hbm, u_hbm, out_hbm, pbuf, buf0, buf1, obuf, sem0, sem1,
                   *, elems_per_worker, chunk_elems):
    c = lax.axis_index("c")
    s = lax.axis_index("s")
    wid = s * _NC + c
    base = pl.multiple_of(wid * elems_per_worker, 8)
    nchunk = elems_per_worker // chunk_elems

    pltpu.sync_copy(p_hbm, pbuf)
    p = pbuf[...]

    bufs = (buf0, buf1)
    sems = (sem0, sem1)
    pltpu.make_async_copy(
        u_hbm.at[pl.ds(base, chunk_elems)], bufs[0], sems[0]).start()

    acc = jnp.zeros((_L,), jnp.float32)
    for k in range(nchunk):
        if k + 1 < nchunk:
            nxt = base + (k + 1) * chunk_elems
            pltpu.make_async_copy(
                u_hbm.at[pl.ds(nxt, chunk_elems)],
                bufs[(k + 1) % 2], sems[(k + 1) % 2]).start()
        pltpu.make_async_copy(
            u_hbm.at[pl.ds(base + k * chunk_elems, chunk_elems)],
            bufs[k % 2], sems[k % 2]).wait()
        buf = bufs[k % 2]

        def row_body(i, a):
            u = buf[pl.ds(i * _L, _L)]
            return a + jnp.where(u < p, 1.0, 0.0)

        acc = lax.fori_loop(0, chunk_elems // _L, row_body, acc, unroll=8)

    obuf[...] = acc
    pltpu.sync_copy(obuf, out_hbm.at[wid])


def _finish_body(m_ref, ops_ref, o_ref, *, s_total):
    m = jnp.sum(m_ref[...], axis=0, keepdims=True)
    v = s_total - 2.0 * m
    o_ref[...] = jnp.sum(ops_ref[...] * v, axis=1, keepdims=True) * (1.0 / s_total)


def kernel(logits, uniforms, ops):
    s_total, n_bits = uniforms.shape
    num_ops = ops.shape[0]
    elems = s_total * n_bits
    epw = elems // _NW
    chunk = 50_000

    probs = jax.nn.sigmoid(logits)
    u1d = uniforms.reshape(elems)

    mesh = plsc.VectorSubcoreMesh(
        core_axis_name="c", subcore_axis_name="s",
        num_cores=_NC, num_subcores=_NS)
    sc_fn = pl.kernel(
        functools.partial(_sc_count_body,
                          elems_per_worker=epw, chunk_elems=chunk),
        out_type=jax.ShapeDtypeStruct((_NW, _L), jnp.float32),
        mesh=mesh,
        scratch_types=[
            pltpu.VMEM((_L,), jnp.float32),
            pltpu.VMEM((chunk,), jnp.float32),
            pltpu.VMEM((chunk,), jnp.float32),
            pltpu.VMEM((_L,), jnp.float32),
            pltpu.SemaphoreType.DMA,
            pltpu.SemaphoreType.DMA,
        ],
    )
    m32 = sc_fn(probs, u1d)

    out = pl.pallas_call(
        functools.partial(_finish_body, s_total=float(s_total)),
        out_shape=jax.ShapeDtypeStruct((num_ops, 1), jnp.float32),
    )(m32, ops)
    return out.reshape(num_ops)


# speedup vs baseline: 1.0442x; 1.0442x over previous
import functools

import jax
import jax.numpy as jnp
from jax import lax
from jax.experimental import pallas as pl
from jax.experimental.pallas import tpu as pltpu
from jax.experimental.pallas import tpu_sc as plsc

_NC = 2
_NS = 16
_NW = _NC * _NS
_L = 16


def _sc_count_body(p_hbm, u_hbm, out_hbm, pbuf, buf0, buf1, obuf, sem0, sem1,
                   *, rows_per_worker, chunk_rows):
    c = lax.axis_index("c")
    s = lax.axis_index("s")
    wid = s * _NC + c
    base = pl.multiple_of(wid * rows_per_worker, 8)
    nchunk = rows_per_worker // chunk_rows

    pltpu.sync_copy(p_hbm, pbuf)
    p = pbuf[...]

    bufs = (buf0, buf1)
    sems = (sem0, sem1)
    pltpu.make_async_copy(
        u_hbm.at[pl.ds(base, chunk_rows)], bufs[0], sems[0]).start()

    acc = jnp.zeros((_L,), jnp.float32)
    for k in range(nchunk):
        if k + 1 < nchunk:
            nxt = base + (k + 1) * chunk_rows
            pltpu.make_async_copy(
                u_hbm.at[pl.ds(nxt, chunk_rows)],
                bufs[(k + 1) % 2], sems[(k + 1) % 2]).start()
        pltpu.make_async_copy(
            u_hbm.at[pl.ds(base + k * chunk_rows, chunk_rows)],
            bufs[k % 2], sems[k % 2]).wait()
        buf = bufs[k % 2]

        def row_body(i, a):
            u = buf[i]
            return a + jnp.where(u < p, 1.0, 0.0)

        acc = lax.fori_loop(0, chunk_rows, row_body, acc, unroll=8)

    obuf[...] = acc
    pltpu.sync_copy(obuf, out_hbm.at[wid])


def _finish_body(m_ref, p_ref, tail_ref, ops_ref, o_ref, *, s_total):
    m = jnp.sum(m_ref[...], axis=0, keepdims=True)
    tail = (tail_ref[...] < p_ref[...]).astype(jnp.float32)
    m = m + jnp.sum(tail, axis=0, keepdims=True)
    v = s_total - 2.0 * m
    o_ref[...] = jnp.sum(ops_ref[...] * v, axis=1, keepdims=True) * (1.0 / s_total)


def kernel(logits, uniforms, ops):
    s_total, n_bits = uniforms.shape
    num_ops = ops.shape[0]
    rpw = (s_total // (8 * _NW)) * 8
    main_rows = rpw * _NW
    tail_rows = s_total - main_rows
    chunk = rpw // 63
    assert rpw % chunk == 0 and chunk % 8 == 0
    assert tail_rows % 8 == 0 and main_rows % tail_rows == 0

    probs = jax.nn.sigmoid(logits)

    mesh = plsc.VectorSubcoreMesh(
        core_axis_name="c", subcore_axis_name="s",
        num_cores=_NC, num_subcores=_NS)
    sc_fn = pl.kernel(
        functools.partial(_sc_count_body,
                          rows_per_worker=rpw, chunk_rows=chunk),
        out_type=jax.ShapeDtypeStruct((_NW, _L), jnp.float32),
        mesh=mesh,
        scratch_types=[
            pltpu.VMEM((_L,), jnp.float32),
            pltpu.VMEM((chunk, n_bits), jnp.float32),
            pltpu.VMEM((chunk, n_bits), jnp.float32),
            pltpu.VMEM((_L,), jnp.float32),
            pltpu.SemaphoreType.DMA,
            pltpu.SemaphoreType.DMA,
        ],
    )
    m32 = sc_fn(probs, uniforms)

    tail_block_idx = main_rows // tail_rows
    out = pl.pallas_call(
        functools.partial(_finish_body, s_total=float(s_total)),
        grid=(1,),
        in_specs=[
            pl.BlockSpec((_NW, n_bits), lambda i: (0, 0)),
            pl.BlockSpec((1, n_bits), lambda i: (0, 0)),
            pl.BlockSpec((tail_rows, n_bits), lambda i: (tail_block_idx, 0)),
            pl.BlockSpec((num_ops, n_bits), lambda i: (0, 0)),
        ],
        out_specs=pl.BlockSpec((num_ops, 1), lambda i: (0, 0)),
        out_shape=jax.ShapeDtypeStruct((num_ops, 1), jnp.float32),
    )(m32, probs.reshape(1, n_bits), uniforms, ops)
    return out.reshape(num_ops)


# speedup vs baseline: 1.0713x; 1.0259x over previous
import functools

import jax
import jax.numpy as jnp
from jax import lax
from jax.experimental import pallas as pl
from jax.experimental.pallas import tpu as pltpu
from jax.experimental.pallas import tpu_sc as plsc

_NC = 2
_NS = 16
_NW = _NC * _NS
_L = 16
_NBUF = 4


def _sc_count_body(p_hbm, u_hbm, out_hbm, *scratch,
                   rows_per_worker, chunk_rows):
    pbuf = scratch[0]
    bufs = scratch[1:1 + _NBUF]
    obuf = scratch[1 + _NBUF]
    sems = scratch[2 + _NBUF:2 + 2 * _NBUF]

    c = lax.axis_index("c")
    s = lax.axis_index("s")
    wid = s * _NC + c
    base = pl.multiple_of(wid * rows_per_worker, 8)
    nchunk = rows_per_worker // chunk_rows

    pltpu.sync_copy(p_hbm, pbuf)
    p = pbuf[...]

    def copy(k):
        return pltpu.make_async_copy(
            u_hbm.at[pl.ds(base + k * chunk_rows, chunk_rows)],
            bufs[k % _NBUF], sems[k % _NBUF])

    for j in range(min(_NBUF - 1, nchunk)):
        copy(j).start()

    acc = jnp.zeros((_L,), jnp.float32)
    for k in range(nchunk):
        if k + _NBUF - 1 < nchunk:
            copy(k + _NBUF - 1).start()
        copy(k).wait()
        buf = bufs[k % _NBUF]

        def row_body(i, a):
            u = buf[i]
            return a + jnp.where(u < p, 1.0, 0.0)

        acc = lax.fori_loop(0, chunk_rows, row_body, acc, unroll=8)

    obuf[...] = acc
    pltpu.sync_copy(obuf, out_hbm.at[wid])


def _finish_body(m_ref, p_ref, tail_ref, ops_ref, o_ref, *, s_total):
    m = jnp.sum(m_ref[...], axis=0, keepdims=True)
    tail = (tail_ref[...] < p_ref[...]).astype(jnp.float32)
    m = m + jnp.sum(tail, axis=0, keepdims=True)
    v = s_total - 2.0 * m
    o_ref[...] = jnp.sum(ops_ref[...] * v, axis=1, keepdims=True) * (1.0 / s_total)


def kernel(logits, uniforms, ops):
    s_total, n_bits = uniforms.shape
    num_ops = ops.shape[0]
    rpw = (s_total // (8 * _NW)) * 8
    main_rows = rpw * _NW
    tail_rows = s_total - main_rows
    chunk_rows = 168
    assert rpw % chunk_rows == 0 and chunk_rows % 8 == 0

    probs = jax.nn.sigmoid(logits)
    u_tail = lax.slice(uniforms, (main_rows, 0), (s_total, n_bits))

    mesh = plsc.VectorSubcoreMesh(
        core_axis_name="c", subcore_axis_name="s",
        num_cores=_NC, num_subcores=_NS)
    sc_fn = pl.kernel(
        functools.partial(_sc_count_body,
                          rows_per_worker=rpw, chunk_rows=chunk_rows),
        out_type=jax.ShapeDtypeStruct((_NW, _L), jnp.float32),
        mesh=mesh,
        scratch_types=(
            [pltpu.VMEM((_L,), jnp.float32)]
            + [pltpu.VMEM((chunk_rows, n_bits), jnp.float32)] * _NBUF
            + [pltpu.VMEM((_L,), jnp.float32)]
            + [pltpu.SemaphoreType.DMA] * _NBUF
        ),
    )
    m32 = sc_fn(probs, uniforms)

    out = pl.pallas_call(
        functools.partial(_finish_body, s_total=float(s_total)),
        out_shape=jax.ShapeDtypeStruct((num_ops, 1), jnp.float32),
    )(m32, probs.reshape(1, n_bits), u_tail, ops)
    return out.reshape(num_ops)
